# baseline (speedup 1.0000x reference)
import jax
import jax.numpy as jnp
from jax import lax
from jax.experimental import pallas as pl
from jax.experimental.pallas import tpu as pltpu


def _match_body(t_ref, pr_ref, loc_ref, bin_ref,
                conf_ref, posf_ref, cebin_ref, lossl_ref, npos_ref,
                *, T, R, P):
    px = pr_ref[0]
    py = pr_ref[1]
    pw = pr_ref[2]
    ph = pr_ref[3]
    pxmin = px - pw * 0.5
    pymin = py - ph * 0.5
    pxmax = px + pw * 0.5
    pymax = py + ph * 0.5
    parea = pw * ph
    pid = (lax.broadcasted_iota(jnp.int32, (R, 128), 0) * 128
           + lax.broadcasted_iota(jnp.int32, (R, 128), 1))

    best_ov = jnp.full((R, 128), -1.0, dtype=jnp.float32)
    best_idx = jnp.zeros((R, 128), dtype=jnp.int32)
    xs1, ys1, xs2, ys2, labs, bps = [], [], [], [], [], []
    for t in range(T):
        x1 = t_ref[0, t, 0]
        y1 = t_ref[0, t, 1]
        x2 = t_ref[0, t, 2]
        y2 = t_ref[0, t, 3]
        lab = t_ref[0, t, 4]
        xs1.append(x1); ys1.append(y1); xs2.append(x2); ys2.append(y2)
        labs.append(lab)
        iw = jnp.maximum(jnp.minimum(x2, pxmax) - jnp.maximum(x1, pxmin), 0.0)
        ih = jnp.maximum(jnp.minimum(y2, pymax) - jnp.maximum(y1, pymin), 0.0)
        inter = iw * ih
        at = (x2 - x1) * (y2 - y1)
        ov = inter / (at + parea - inter)
        upd = ov > best_ov
        best_idx = jnp.where(upd, t, best_idx)
        best_ov = jnp.where(upd, ov, best_ov)
        m = jnp.max(ov)
        bps.append(jnp.min(jnp.where(ov == m, pid, P)))
    for t in range(T):
        hit = pid == bps[t]
        best_ov = jnp.where(hit, 2.0, best_ov)
        best_idx = jnp.where(hit, t, best_idx)
    conf_i = jnp.zeros((R, 128), dtype=jnp.int32)
    mx1 = jnp.zeros((R, 128), dtype=jnp.float32)
    my1 = jnp.zeros((R, 128), dtype=jnp.float32)
    mx2 = jnp.zeros((R, 128), dtype=jnp.float32)
    my2 = jnp.zeros((R, 128), dtype=jnp.float32)
    for t in range(T):
        sel = best_idx == t
        conf_i = jnp.where(sel, labs[t].astype(jnp.int32) + 1, conf_i)
        mx1 = jnp.where(sel, xs1[t], mx1)
        my1 = jnp.where(sel, ys1[t], my1)
        mx2 = jnp.where(sel, xs2[t], mx2)
        my2 = jnp.where(sel, ys2[t], my2)
    conf_i = jnp.where(best_ov < 0.5, 0, conf_i)
    pos = conf_i > 0
    posf = pos.astype(jnp.float32)
    conf_ref[0] = conf_i
    posf_ref[0] = posf
    npos_ref[0] = jnp.sum(posf).reshape(1, 1)

    g1 = ((mx1 + mx2) * 0.5 - px) / (0.1 * pw)
    g2 = ((my1 + my2) * 0.5 - py) / (0.1 * ph)
    g3 = jnp.log((mx2 - mx1) / pw) / 0.2
    g4 = jnp.log((my2 - my1) / ph) / 0.2
    acc = jnp.float32(0.0)
    for c, g in enumerate((g1, g2, g3, g4)):
        d = loc_ref[0, c] - g
        ad = jnp.abs(d)
        sl = jnp.where(ad < 1.0, 0.5 * d * d, ad - 0.5)
        acc = acc + jnp.sum(jnp.where(pos, sl, 0.0))
    lossl_ref[0] = acc.reshape(1, 1)

    b0 = bin_ref[0, 0]
    b1 = bin_ref[0, 1]
    m2 = jnp.maximum(b0, b1)
    lse = m2 + jnp.log(jnp.exp(b0 - m2) + jnp.exp(b1 - m2))
    cebin_ref[0] = lse - jnp.where(pos, b1, b0)


def _conf_body(x_ref, s_ref, ct_ref, cec_ref, s0_ref, pd_ref, *, BP, C):
    x = x_ref[0]
    ct = ct_ref[0, 0]
    ones_c = jnp.ones((C, 1), dtype=jnp.float32)
    m = jnp.max(x, axis=1, keepdims=True)
    e = jnp.exp(x - m)
    se = jnp.dot(e, ones_c, preferred_element_type=jnp.float32,
                 precision=lax.Precision.HIGHEST)
    lse = m + jnp.log(se)
    cio = lax.broadcasted_iota(jnp.int32, (1, C), 1)
    oh = cio == ct
    s = s_ref[0]
    tgt = jnp.dot(jnp.where(oh, x, 0.0), ones_c,
                  preferred_element_type=jnp.float32,
                  precision=lax.Precision.HIGHEST)
    g = jnp.dot(jnp.where(oh, s, 0.0), ones_c,
                preferred_element_type=jnp.float32,
                precision=lax.Precision.HIGHEST)
    cec_ref[0, 0] = lse - tgt
    s0_ref[0, 0] = s[:, 0:1]
    pd_ref[0, 0] = jnp.sum(jnp.where(ct != 0, g, 0.0)).reshape(1, 1)


def _mine_body(cb_ref, cc_ref, pf_ref, s0_ref, np_ref,
               nb_ref, nm_ref, lb_ref, lc_ref, nd_ref, *, R, P):
    posf = pf_ref[0]
    pos = posf > 0.5
    npos_s = jnp.sum(np_ref[0])
    k = jnp.minimum(npos_s * 3.0, jnp.float32(P - 1)).astype(jnp.int32)
    ceb = cb_ref[0]
    cec = cc_ref[0]

    ir = lax.broadcasted_iota(jnp.int32, (R, R), 0)
    jr = lax.broadcasted_iota(jnp.int32, (R, R), 1)
    Ls = (jr < ir).astype(jnp.float32)
    ia = lax.broadcasted_iota(jnp.int32, (128, 128), 0)
    ja = lax.broadcasted_iota(jnp.int32, (128, 128), 1)
    Us = (ia < ja).astype(jnp.float32)

    def mine(ce):
        loss = jnp.maximum(jnp.where(pos, 0.0, ce), 0.0)
        keys = lax.bitcast_convert_type(loss, jnp.int32)

        def body(_, carry):
            lo, hi = carry
            d = hi - lo
            mid = lo + (d >> 1) + (d & 1)
            cnt = jnp.sum((keys >= mid).astype(jnp.int32))
            pred = cnt >= k
            return (jnp.where(pred, mid, lo), jnp.where(pred, hi, mid - 1))

        lo, _ = lax.fori_loop(0, 31, body,
                              (jnp.int32(0), jnp.int32(2**31 - 1)))
        gt = keys > lo
        cgt = jnp.sum(gt.astype(jnp.int32))
        eq = keys == lo
        eqf = eq.astype(jnp.float32)
        rows = jnp.sum(eqf, axis=1, keepdims=True)
        row_off = jnp.dot(Ls, rows, preferred_element_type=jnp.float32)
        lane_cum = jnp.dot(eqf, Us, preferred_element_type=jnp.float32)
        cum = row_off + lane_cum
        extra = (k - cgt).astype(jnp.float32)
        return gt | (eq & (cum < extra))

    negb = mine(ceb)
    negm = mine(cec)
    nb_ref[0] = negb.astype(jnp.float32)
    nm_ref[0] = negm.astype(jnp.float32)
    lb_ref[0] = jnp.sum(jnp.where(pos | negb, ceb, 0.0)).reshape(1, 1)
    lc_ref[0] = jnp.sum(jnp.where(pos | negm, cec, 0.0)).reshape(1, 1)
    nd_ref[0] = jnp.sum(jnp.where(negm, s0_ref[0], 0.0)).reshape(1, 1)


def kernel(loc_data, conf_data, bin_conf_data, priors, targets,
           conf_target_stand_dist):
    import functools
    N_IMG, P, C = conf_data.shape
    T = targets.shape[1]
    R = P // 128
    BP = min(2048, P)
    NB = P // BP

    pr_t = priors.T.reshape(4, R, 128)
    loc_t = loc_data.transpose(0, 2, 1).reshape(N_IMG, 4, R, 128)
    bin_t = bin_conf_data.transpose(0, 2, 1).reshape(N_IMG, 2, R, 128)

    f32 = jnp.float32
    conf_i, posf, cebin, lossl, nposs = pl.pallas_call(
        functools.partial(_match_body, T=T, R=R, P=P),
        grid=(N_IMG,),
        in_specs=[
            pl.BlockSpec((1, T, 5), lambda i: (i, 0, 0),
                         memory_space=pltpu.SMEM),
            pl.BlockSpec((4, R, 128), lambda i: (0, 0, 0)),
            pl.BlockSpec((1, 4, R, 128), lambda i: (i, 0, 0, 0)),
            pl.BlockSpec((1, 2, R, 128), lambda i: (i, 0, 0, 0)),
        ],
        out_specs=[
            pl.BlockSpec((1, R, 128), lambda i: (i, 0, 0)),
            pl.BlockSpec((1, R, 128), lambda i: (i, 0, 0)),
            pl.BlockSpec((1, R, 128), lambda i: (i, 0, 0)),
            pl.BlockSpec((1, 1, 1), lambda i: (i, 0, 0)),
            pl.BlockSpec((1, 1, 1), lambda i: (i, 0, 0)),
        ],
        out_shape=[
            jax.ShapeDtypeStruct((N_IMG, R, 128), jnp.int32),
            jax.ShapeDtypeStruct((N_IMG, R, 128), f32),
            jax.ShapeDtypeStruct((N_IMG, R, 128), f32),
            jax.ShapeDtypeStruct((N_IMG, 1, 1), f32),
            jax.ShapeDtypeStruct((N_IMG, 1, 1), f32),
        ],
        compiler_params=pltpu.CompilerParams(
            dimension_semantics=("parallel",)),
    )(targets, pr_t, loc_t, bin_t)

    ct4 = conf_i.reshape(N_IMG, NB, BP, 1)
    cec4, s04, pd4 = pl.pallas_call(
        functools.partial(_conf_body, BP=BP, C=C),
        grid=(N_IMG, NB),
        in_specs=[
            pl.BlockSpec((1, BP, C), lambda n, j: (n, j, 0)),
            pl.BlockSpec((1, BP, C), lambda n, j: (n, j, 0)),
            pl.BlockSpec((1, 1, BP, 1), lambda n, j: (n, j, 0, 0)),
        ],
        out_specs=[
            pl.BlockSpec((1, 1, BP, 1), lambda n, j: (n, j, 0, 0)),
            pl.BlockSpec((1, 1, BP, 1), lambda n, j: (n, j, 0, 0)),
            pl.BlockSpec((1, 1, 1, 1), lambda n, j: (n, j, 0, 0)),
        ],
        out_shape=[
            jax.ShapeDtypeStruct((N_IMG, NB, BP, 1), f32),
            jax.ShapeDtypeStruct((N_IMG, NB, BP, 1), f32),
            jax.ShapeDtypeStruct((N_IMG, NB, 1, 1), f32),
        ],
        compiler_params=pltpu.CompilerParams(
            dimension_semantics=("parallel", "parallel")),
    )(conf_data, conf_target_stand_dist, ct4)

    cec = cec4.reshape(N_IMG, R, 128)
    s0 = s04.reshape(N_IMG, R, 128)

    nbf, nmf, lb, lc, nd = pl.pallas_call(
        functools.partial(_mine_body, R=R, P=P),
        grid=(N_IMG,),
        in_specs=[
            pl.BlockSpec((1, R, 128), lambda i: (i, 0, 0)),
            pl.BlockSpec((1, R, 128), lambda i: (i, 0, 0)),
            pl.BlockSpec((1, R, 128), lambda i: (i, 0, 0)),
            pl.BlockSpec((1, R, 128), lambda i: (i, 0, 0)),
            pl.BlockSpec((1, 1, 1), lambda i: (i, 0, 0)),
        ],
        out_specs=[
            pl.BlockSpec((1, R, 128), lambda i: (i, 0, 0)),
            pl.BlockSpec((1, R, 128), lambda i: (i, 0, 0)),
            pl.BlockSpec((1, 1, 1), lambda i: (i, 0, 0)),
            pl.BlockSpec((1, 1, 1), lambda i: (i, 0, 0)),
            pl.BlockSpec((1, 1, 1), lambda i: (i, 0, 0)),
        ],
        out_shape=[
            jax.ShapeDtypeStruct((N_IMG, R, 128), f32),
            jax.ShapeDtypeStruct((N_IMG, R, 128), f32),
            jax.ShapeDtypeStruct((N_IMG, 1, 1), f32),
            jax.ShapeDtypeStruct((N_IMG, 1, 1), f32),
            jax.ShapeDtypeStruct((N_IMG, 1, 1), f32),
        ],
        compiler_params=pltpu.CompilerParams(
            dimension_semantics=("parallel",)),
    )(cebin, cec, posf, s0, nposs)

    N = jnp.maximum(jnp.sum(nposs), 1.0)
    loss_l = jnp.sum(lossl) / N
    loss_c = jnp.sum(lc) / N
    loss_bin = jnp.sum(lb) / N
    dist_loss = (-jnp.sum(pd4) + 0.2 * jnp.sum(nd)) / N
    pos_out = posf.reshape(N_IMG, P) > 0.5
    negb_out = nbf.reshape(N_IMG, P) > 0.5
    negm_out = nmf.reshape(N_IMG, P) > 0.5
    return (loss_l, loss_c, loss_bin, dist_loss, pos_out, negb_out, negm_out)


# speedup vs baseline: 1.1883x; 1.1883x over previous
import jax
import jax.numpy as jnp
from jax import lax
from jax.experimental import pallas as pl
from jax.experimental.pallas import tpu as pltpu


def _match_body(t_ref, pr_ref, loc_ref, bin_ref,
                conf_ref, posf_ref, cebin_ref, lossl_ref, npos_ref,
                *, T, R, P):
    px = pr_ref[0]
    py = pr_ref[1]
    pw = pr_ref[2]
    ph = pr_ref[3]
    pxmin = px - pw * 0.5
    pymin = py - ph * 0.5
    pxmax = px + pw * 0.5
    pymax = py + ph * 0.5
    parea = pw * ph
    pid = (lax.broadcasted_iota(jnp.int32, (R, 128), 0) * 128
           + lax.broadcasted_iota(jnp.int32, (R, 128), 1))

    best_ov = jnp.full((R, 128), -1.0, dtype=jnp.float32)
    best_idx = jnp.zeros((R, 128), dtype=jnp.int32)
    xs1, ys1, xs2, ys2, labs, bps = [], [], [], [], [], []
    for t in range(T):
        x1 = t_ref[0, t, 0]
        y1 = t_ref[0, t, 1]
        x2 = t_ref[0, t, 2]
        y2 = t_ref[0, t, 3]
        lab = t_ref[0, t, 4]
        xs1.append(x1); ys1.append(y1); xs2.append(x2); ys2.append(y2)
        labs.append(lab)
        iw = jnp.maximum(jnp.minimum(x2, pxmax) - jnp.maximum(x1, pxmin), 0.0)
        ih = jnp.maximum(jnp.minimum(y2, pymax) - jnp.maximum(y1, pymin), 0.0)
        inter = iw * ih
        at = (x2 - x1) * (y2 - y1)
        ov = inter / (at + parea - inter)
        upd = ov > best_ov
        best_idx = jnp.where(upd, t, best_idx)
        best_ov = jnp.where(upd, ov, best_ov)
        m = jnp.max(ov)
        bps.append(jnp.min(jnp.where(ov == m, pid, P)))
    force_t = jnp.full((R, 128), -1, dtype=jnp.int32)
    for t in range(T):
        force_t = jnp.where(pid == bps[t], t, force_t)
    forced = force_t >= 0
    best_ov = jnp.where(forced, 2.0, best_ov)
    best_idx = jnp.where(forced, force_t, best_idx)
    conf_i = jnp.zeros((R, 128), dtype=jnp.int32)
    mx1 = jnp.zeros((R, 128), dtype=jnp.float32)
    my1 = jnp.zeros((R, 128), dtype=jnp.float32)
    mx2 = jnp.zeros((R, 128), dtype=jnp.float32)
    my2 = jnp.zeros((R, 128), dtype=jnp.float32)
    for t in range(T):
        sel = best_idx == t
        conf_i = jnp.where(sel, labs[t].astype(jnp.int32) + 1, conf_i)
        mx1 = jnp.where(sel, xs1[t], mx1)
        my1 = jnp.where(sel, ys1[t], my1)
        mx2 = jnp.where(sel, xs2[t], mx2)
        my2 = jnp.where(sel, ys2[t], my2)
    conf_i = jnp.where(best_ov < 0.5, 0, conf_i)
    pos = conf_i > 0
    posf = pos.astype(jnp.float32)
    conf_ref[0] = conf_i
    posf_ref[0] = posf
    npos_ref[0] = jnp.sum(posf).reshape(1, 1)

    g1 = ((mx1 + mx2) * 0.5 - px) / (0.1 * pw)
    g2 = ((my1 + my2) * 0.5 - py) / (0.1 * ph)
    g3 = jnp.log((mx2 - mx1) / pw) / 0.2
    g4 = jnp.log((my2 - my1) / ph) / 0.2
    acc = jnp.float32(0.0)
    for c, g in enumerate((g1, g2, g3, g4)):
        d = loc_ref[0, c] - g
        ad = jnp.abs(d)
        sl = jnp.where(ad < 1.0, 0.5 * d * d, ad - 0.5)
        acc = acc + jnp.sum(jnp.where(pos, sl, 0.0))
    lossl_ref[0] = acc.reshape(1, 1)

    b0 = bin_ref[0, 0]
    b1 = bin_ref[0, 1]
    m2 = jnp.maximum(b0, b1)
    lse = m2 + jnp.log(jnp.exp(b0 - m2) + jnp.exp(b1 - m2))
    cebin_ref[0] = lse - jnp.where(pos, b1, b0)


def _conf_body(x_ref, s_ref, ct_ref, cec_ref, s0_ref, pd_ref, *, BP, C):
    x = x_ref[0]
    ct = ct_ref[0, 0]
    m = jnp.max(x, axis=1, keepdims=True)
    e = jnp.exp(x - m)
    lse = m + jnp.log(jnp.sum(e, axis=1, keepdims=True))
    cio = lax.broadcasted_iota(jnp.int32, (1, C), 1)
    oh = cio == ct
    s = s_ref[0]
    tgt = jnp.sum(jnp.where(oh, x, 0.0), axis=1, keepdims=True)
    g = jnp.sum(jnp.where(oh, s, 0.0), axis=1, keepdims=True)
    cec_ref[0, 0] = lse - tgt
    s0_ref[0, 0] = s[:, 0:1]
    pd_ref[0, 0] = jnp.sum(jnp.where(ct != 0, g, 0.0)).reshape(1, 1)


def _mine_body(cb_ref, cc_ref, pf_ref, s0_ref, np_ref,
               nb_ref, nm_ref, lb_ref, lc_ref, nd_ref, *, N_IMG, R, P):
    posf = pf_ref[...]
    pos = posf > 0.5
    npos = np_ref[...]
    k = jnp.minimum(npos * 3.0, jnp.float32(P - 1)).astype(jnp.int32)
    ceb = cb_ref[...]
    cec = cc_ref[...]

    ir = lax.broadcasted_iota(jnp.int32, (R, R), 0)
    jr = lax.broadcasted_iota(jnp.int32, (R, R), 1)
    UsR = (ir < jr).astype(jnp.float32)
    ia = lax.broadcasted_iota(jnp.int32, (128, 128), 0)
    ja = lax.broadcasted_iota(jnp.int32, (128, 128), 1)
    Us = (ia < ja).astype(jnp.float32)

    def per_image_sum(x):
        return jnp.sum(jnp.sum(x, axis=2), axis=1, keepdims=True)[:, :, None]

    def mine(ce):
        loss = jnp.maximum(jnp.where(pos, 0.0, ce), 0.0)
        keys = lax.bitcast_convert_type(loss, jnp.int32)

        def body(_, carry):
            lo, hi = carry
            d = hi - lo
            mid = lo + (d >> 1) + (d & 1)
            cnt = per_image_sum((keys >= mid).astype(jnp.int32))
            pred = cnt >= k
            return (jnp.where(pred, mid, lo), jnp.where(pred, hi, mid - 1))

        lo0 = jnp.zeros((N_IMG, 1, 1), jnp.int32)
        hi0 = jnp.full((N_IMG, 1, 1), 2**31 - 1, jnp.int32)
        lo, _ = lax.fori_loop(0, 31, body, (lo0, hi0))
        gt = keys > lo
        cgt = per_image_sum(gt.astype(jnp.int32))
        eq = keys == lo
        eqf = eq.astype(jnp.float32)
        rows = jnp.sum(eqf, axis=2)
        row_off = jnp.dot(rows, UsR, preferred_element_type=jnp.float32)
        lane_cum = lax.dot_general(eqf, Us, (((2,), (0,)), ((), ())),
                                   preferred_element_type=jnp.float32)
        cum = row_off[:, :, None] + lane_cum
        extra = (k - cgt).astype(jnp.float32)
        return gt | (eq & (cum < extra))

    negb = mine(ceb)
    negm = mine(cec)
    nb_ref[...] = negb.astype(jnp.float32)
    nm_ref[...] = negm.astype(jnp.float32)
    lb_ref[...] = per_image_sum(jnp.where(pos | negb, ceb, 0.0))
    lc_ref[...] = per_image_sum(jnp.where(pos | negm, cec, 0.0))
    nd_ref[...] = per_image_sum(jnp.where(negm, s0_ref[...], 0.0))


def kernel(loc_data, conf_data, bin_conf_data, priors, targets,
           conf_target_stand_dist):
    import functools
    N_IMG, P, C = conf_data.shape
    T = targets.shape[1]
    R = P // 128
    BP = min(2048, P)
    NB = P // BP

    pr_t = priors.T.reshape(4, R, 128)
    loc_t = loc_data.transpose(0, 2, 1).reshape(N_IMG, 4, R, 128)
    bin_t = bin_conf_data.transpose(0, 2, 1).reshape(N_IMG, 2, R, 128)

    f32 = jnp.float32
    conf_i, posf, cebin, lossl, nposs = pl.pallas_call(
        functools.partial(_match_body, T=T, R=R, P=P),
        grid=(N_IMG,),
        in_specs=[
            pl.BlockSpec((1, T, 5), lambda i: (i, 0, 0),
                         memory_space=pltpu.SMEM),
            pl.BlockSpec((4, R, 128), lambda i: (0, 0, 0)),
            pl.BlockSpec((1, 4, R, 128), lambda i: (i, 0, 0, 0)),
            pl.BlockSpec((1, 2, R, 128), lambda i: (i, 0, 0, 0)),
        ],
        out_specs=[
            pl.BlockSpec((1, R, 128), lambda i: (i, 0, 0)),
            pl.BlockSpec((1, R, 128), lambda i: (i, 0, 0)),
            pl.BlockSpec((1, R, 128), lambda i: (i, 0, 0)),
            pl.BlockSpec((1, 1, 1), lambda i: (i, 0, 0)),
            pl.BlockSpec((1, 1, 1), lambda i: (i, 0, 0)),
        ],
        out_shape=[
            jax.ShapeDtypeStruct((N_IMG, R, 128), jnp.int32),
            jax.ShapeDtypeStruct((N_IMG, R, 128), f32),
            jax.ShapeDtypeStruct((N_IMG, R, 128), f32),
            jax.ShapeDtypeStruct((N_IMG, 1, 1), f32),
            jax.ShapeDtypeStruct((N_IMG, 1, 1), f32),
        ],
    )(targets, pr_t, loc_t, bin_t)

    ct4 = conf_i.reshape(N_IMG, NB, BP, 1)
    cec4, s04, pd4 = pl.pallas_call(
        functools.partial(_conf_body, BP=BP, C=C),
        grid=(N_IMG, NB),
        in_specs=[
            pl.BlockSpec((1, BP, C), lambda n, j: (n, j, 0)),
            pl.BlockSpec((1, BP, C), lambda n, j: (n, j, 0)),
            pl.BlockSpec((1, 1, BP, 1), lambda n, j: (n, j, 0, 0)),
        ],
        out_specs=[
            pl.BlockSpec((1, 1, BP, 1), lambda n, j: (n, j, 0, 0)),
            pl.BlockSpec((1, 1, BP, 1), lambda n, j: (n, j, 0, 0)),
            pl.BlockSpec((1, 1, 1, 1), lambda n, j: (n, j, 0, 0)),
        ],
        out_shape=[
            jax.ShapeDtypeStruct((N_IMG, NB, BP, 1), f32),
            jax.ShapeDtypeStruct((N_IMG, NB, BP, 1), f32),
            jax.ShapeDtypeStruct((N_IMG, NB, 1, 1), f32),
        ],
    )(conf_data, conf_target_stand_dist, ct4)

    cec = cec4.reshape(N_IMG, R, 128)
    s0 = s04.reshape(N_IMG, R, 128)

    nbf, nmf, lb, lc, nd = pl.pallas_call(
        functools.partial(_mine_body, N_IMG=N_IMG, R=R, P=P),
        out_shape=[
            jax.ShapeDtypeStruct((N_IMG, R, 128), f32),
            jax.ShapeDtypeStruct((N_IMG, R, 128), f32),
            jax.ShapeDtypeStruct((N_IMG, 1, 1), f32),
            jax.ShapeDtypeStruct((N_IMG, 1, 1), f32),
            jax.ShapeDtypeStruct((N_IMG, 1, 1), f32),
        ],
    )(cebin, cec, posf, s0, nposs)

    N = jnp.maximum(jnp.sum(nposs), 1.0)
    loss_l = jnp.sum(lossl) / N
    loss_c = jnp.sum(lc) / N
    loss_bin = jnp.sum(lb) / N
    dist_loss = (-jnp.sum(pd4) + 0.2 * jnp.sum(nd)) / N
    pos_out = posf.reshape(N_IMG, P) > 0.5
    negb_out = nbf.reshape(N_IMG, P) > 0.5
    negm_out = nmf.reshape(N_IMG, P) > 0.5
    return (loss_l, loss_c, loss_bin, dist_loss, pos_out, negb_out, negm_out)


# speedup vs baseline: 1.2493x; 1.0513x over previous
import jax
import jax.numpy as jnp
from jax import lax
from jax.experimental import pallas as pl
from jax.experimental.pallas import tpu as pltpu


def _match_body(t_ref, pr_ref, loc_ref, bin_ref,
                conf_ref, posf_ref, cebin_ref, lossl_ref, npos_ref,
                *, T, R, P):
    px = pr_ref[0]
    py = pr_ref[1]
    pw = pr_ref[2]
    ph = pr_ref[3]
    pxmin = px - pw * 0.5
    pymin = py - ph * 0.5
    pxmax = px + pw * 0.5
    pymax = py + ph * 0.5
    parea = pw * ph
    pid = (lax.broadcasted_iota(jnp.int32, (R, 128), 0) * 128
           + lax.broadcasted_iota(jnp.int32, (R, 128), 1))

    best_ov = jnp.full((R, 128), -1.0, dtype=jnp.float32)
    best_idx = jnp.zeros((R, 128), dtype=jnp.int32)
    xs1, ys1, xs2, ys2, labs, bps = [], [], [], [], [], []
    for t in range(T):
        x1 = t_ref[0, t, 0]
        y1 = t_ref[0, t, 1]
        x2 = t_ref[0, t, 2]
        y2 = t_ref[0, t, 3]
        lab = t_ref[0, t, 4]
        xs1.append(x1); ys1.append(y1); xs2.append(x2); ys2.append(y2)
        labs.append(lab)
        iw = jnp.maximum(jnp.minimum(x2, pxmax) - jnp.maximum(x1, pxmin), 0.0)
        ih = jnp.maximum(jnp.minimum(y2, pymax) - jnp.maximum(y1, pymin), 0.0)
        inter = iw * ih
        at = (x2 - x1) * (y2 - y1)
        ov = inter / (at + parea - inter)
        upd = ov > best_ov
        best_idx = jnp.where(upd, t, best_idx)
        best_ov = jnp.where(upd, ov, best_ov)
        m = jnp.max(ov)
        bps.append(jnp.min(jnp.where(ov == m, pid, P)))
    force_t = jnp.full((R, 128), -1, dtype=jnp.int32)
    for t in range(T):
        force_t = jnp.where(pid == bps[t], t, force_t)
    forced = force_t >= 0
    best_ov = jnp.where(forced, 2.0, best_ov)
    best_idx = jnp.where(forced, force_t, best_idx)
    conf_i = jnp.zeros((R, 128), dtype=jnp.int32)
    mx1 = jnp.zeros((R, 128), dtype=jnp.float32)
    my1 = jnp.zeros((R, 128), dtype=jnp.float32)
    mx2 = jnp.zeros((R, 128), dtype=jnp.float32)
    my2 = jnp.zeros((R, 128), dtype=jnp.float32)
    for t in range(T):
        sel = best_idx == t
        conf_i = jnp.where(sel, labs[t].astype(jnp.int32) + 1, conf_i)
        mx1 = jnp.where(sel, xs1[t], mx1)
        my1 = jnp.where(sel, ys1[t], my1)
        mx2 = jnp.where(sel, xs2[t], mx2)
        my2 = jnp.where(sel, ys2[t], my2)
    conf_i = jnp.where(best_ov < 0.5, 0, conf_i)
    pos = conf_i > 0
    posf = pos.astype(jnp.float32)
    conf_ref[0] = conf_i
    posf_ref[0] = posf
    npos_ref[0] = jnp.sum(posf).reshape(1, 1)

    g1 = ((mx1 + mx2) * 0.5 - px) / (0.1 * pw)
    g2 = ((my1 + my2) * 0.5 - py) / (0.1 * ph)
    g3 = jnp.log((mx2 - mx1) / pw) / 0.2
    g4 = jnp.log((my2 - my1) / ph) / 0.2
    acc = jnp.float32(0.0)
    for c, g in enumerate((g1, g2, g3, g4)):
        d = loc_ref[0, c] - g
        ad = jnp.abs(d)
        sl = jnp.where(ad < 1.0, 0.5 * d * d, ad - 0.5)
        acc = acc + jnp.sum(jnp.where(pos, sl, 0.0))
    lossl_ref[0] = acc.reshape(1, 1)

    b0 = bin_ref[0, 0]
    b1 = bin_ref[0, 1]
    m2 = jnp.maximum(b0, b1)
    lse = m2 + jnp.log(jnp.exp(b0 - m2) + jnp.exp(b1 - m2))
    cebin_ref[0] = lse - jnp.where(pos, b1, b0)


def _conf_body(x_ref, s_ref, ct_ref, cec_ref, s0_ref, pd_ref, *, BP, C):
    x = x_ref[0]
    ct = ct_ref[0, 0]
    m = jnp.max(x, axis=1, keepdims=True)
    e = jnp.exp(x - m)
    lse = m + jnp.log(jnp.sum(e, axis=1, keepdims=True))
    cio = lax.broadcasted_iota(jnp.int32, (1, C), 1)
    oh = cio == ct
    s = s_ref[0]
    tgt = jnp.sum(jnp.where(oh, x, 0.0), axis=1, keepdims=True)
    g = jnp.sum(jnp.where(oh, s, 0.0), axis=1, keepdims=True)
    cec_ref[0, 0] = lse - tgt
    s0_ref[0, 0] = s[:, 0:1]
    pd_ref[0, 0] = jnp.sum(jnp.where(ct != 0, g, 0.0)).reshape(1, 1)


def _mine_body(cb_ref, cc_ref, pf_ref, s0_ref, np_ref,
               nb_ref, nm_ref, lb_ref, lc_ref, nd_ref, *, N_IMG, R, P):
    posf = pf_ref[...]
    pos = posf > 0.5
    npos = np_ref[...]
    k = jnp.minimum(npos * 3.0, jnp.float32(P - 1)).astype(jnp.int32)
    ceb = cb_ref[...]
    cec = cc_ref[...]

    ir = lax.broadcasted_iota(jnp.int32, (R, R), 0)
    jr = lax.broadcasted_iota(jnp.int32, (R, R), 1)
    UsR = (ir < jr).astype(jnp.float32)
    ia = lax.broadcasted_iota(jnp.int32, (128, 128), 0)
    ja = lax.broadcasted_iota(jnp.int32, (128, 128), 1)
    Us = (ia < ja).astype(jnp.float32)

    def per_image_sum(x):
        return jnp.sum(jnp.sum(x, axis=2), axis=1, keepdims=True)[:, :, None]

    def mine(ce):
        loss = jnp.maximum(jnp.where(pos, 0.0, ce), 0.0)
        keys = lax.bitcast_convert_type(loss, jnp.int32)

        def body(_, carry):
            lo, hi = carry
            d = hi - lo
            mid = lo + (d >> 1) + (d & 1)
            cnt = per_image_sum((keys >= mid).astype(jnp.int32))
            pred = cnt >= k
            return (jnp.where(pred, mid, lo), jnp.where(pred, hi, mid - 1))

        lo0 = jnp.zeros((N_IMG, 1, 1), jnp.int32)
        hi0 = jnp.full((N_IMG, 1, 1), 2**31 - 1, jnp.int32)
        lo, _ = lax.fori_loop(0, 31, body, (lo0, hi0))
        gt = keys > lo
        cgt = per_image_sum(gt.astype(jnp.int32))
        eq = keys == lo
        eqf = eq.astype(jnp.float32)
        rows = jnp.sum(eqf, axis=2)
        row_off = jnp.dot(rows, UsR, preferred_element_type=jnp.float32)
        lane_cum = lax.dot_general(eqf, Us, (((2,), (0,)), ((), ())),
                                   preferred_element_type=jnp.float32)
        cum = row_off[:, :, None] + lane_cum
        extra = (k - cgt).astype(jnp.float32)
        return gt | (eq & (cum < extra))

    negb = mine(ceb)
    negm = mine(cec)
    nb_ref[...] = negb.astype(jnp.float32)
    nm_ref[...] = negm.astype(jnp.float32)
    lb_ref[...] = per_image_sum(jnp.where(pos | negb, ceb, 0.0))
    lc_ref[...] = per_image_sum(jnp.where(pos | negm, cec, 0.0))
    nd_ref[...] = per_image_sum(jnp.where(negm, s0_ref[...], 0.0))


def kernel(loc_data, conf_data, bin_conf_data, priors, targets,
           conf_target_stand_dist):
    import functools
    N_IMG, P, C = conf_data.shape
    T = targets.shape[1]
    R = P // 128
    BP = min(4096, P)
    NB = P // BP

    pr_t = priors.T.reshape(4, R, 128)
    loc_t = loc_data.transpose(0, 2, 1).reshape(N_IMG, 4, R, 128)
    bin_t = bin_conf_data.transpose(0, 2, 1).reshape(N_IMG, 2, R, 128)

    f32 = jnp.float32
    conf_i, posf, cebin, lossl, nposs = pl.pallas_call(
        functools.partial(_match_body, T=T, R=R, P=P),
        grid=(N_IMG,),
        in_specs=[
            pl.BlockSpec((1, T, 5), lambda i: (i, 0, 0),
                         memory_space=pltpu.SMEM),
            pl.BlockSpec((4, R, 128), lambda i: (0, 0, 0)),
            pl.BlockSpec((1, 4, R, 128), lambda i: (i, 0, 0, 0)),
            pl.BlockSpec((1, 2, R, 128), lambda i: (i, 0, 0, 0)),
        ],
        out_specs=[
            pl.BlockSpec((1, R, 128), lambda i: (i, 0, 0)),
            pl.BlockSpec((1, R, 128), lambda i: (i, 0, 0)),
            pl.BlockSpec((1, R, 128), lambda i: (i, 0, 0)),
            pl.BlockSpec((1, 1, 1), lambda i: (i, 0, 0)),
            pl.BlockSpec((1, 1, 1), lambda i: (i, 0, 0)),
        ],
        out_shape=[
            jax.ShapeDtypeStruct((N_IMG, R, 128), jnp.int32),
            jax.ShapeDtypeStruct((N_IMG, R, 128), f32),
            jax.ShapeDtypeStruct((N_IMG, R, 128), f32),
            jax.ShapeDtypeStruct((N_IMG, 1, 1), f32),
            jax.ShapeDtypeStruct((N_IMG, 1, 1), f32),
        ],
    )(targets, pr_t, loc_t, bin_t)

    ct4 = conf_i.reshape(N_IMG, NB, BP, 1)
    cec4, s04, pd4 = pl.pallas_call(
        functools.partial(_conf_body, BP=BP, C=C),
        grid=(N_IMG, NB),
        in_specs=[
            pl.BlockSpec((1, BP, C), lambda n, j: (n, j, 0)),
            pl.BlockSpec((1, BP, C), lambda n, j: (n, j, 0)),
            pl.BlockSpec((1, 1, BP, 1), lambda n, j: (n, j, 0, 0)),
        ],
        out_specs=[
            pl.BlockSpec((1, 1, BP, 1), lambda n, j: (n, j, 0, 0)),
            pl.BlockSpec((1, 1, BP, 1), lambda n, j: (n, j, 0, 0)),
            pl.BlockSpec((1, 1, 1, 1), lambda n, j: (n, j, 0, 0)),
        ],
        out_shape=[
            jax.ShapeDtypeStruct((N_IMG, NB, BP, 1), f32),
            jax.ShapeDtypeStruct((N_IMG, NB, BP, 1), f32),
            jax.ShapeDtypeStruct((N_IMG, NB, 1, 1), f32),
        ],
    )(conf_data, conf_target_stand_dist, ct4)

    cec = cec4.reshape(N_IMG, R, 128)
    s0 = s04.reshape(N_IMG, R, 128)

    nbf, nmf, lb, lc, nd = pl.pallas_call(
        functools.partial(_mine_body, N_IMG=N_IMG, R=R, P=P),
        out_shape=[
            jax.ShapeDtypeStruct((N_IMG, R, 128), f32),
            jax.ShapeDtypeStruct((N_IMG, R, 128), f32),
            jax.ShapeDtypeStruct((N_IMG, 1, 1), f32),
            jax.ShapeDtypeStruct((N_IMG, 1, 1), f32),
            jax.ShapeDtypeStruct((N_IMG, 1, 1), f32),
        ],
    )(cebin, cec, posf, s0, nposs)

    N = jnp.maximum(jnp.sum(nposs), 1.0)
    loss_l = jnp.sum(lossl) / N
    loss_c = jnp.sum(lc) / N
    loss_bin = jnp.sum(lb) / N
    dist_loss = (-jnp.sum(pd4) + 0.2 * jnp.sum(nd)) / N
    pos_out = posf.reshape(N_IMG, P) > 0.5
    negb_out = nbf.reshape(N_IMG, P) > 0.5
    negm_out = nmf.reshape(N_IMG, P) > 0.5
    return (loss_l, loss_c, loss_bin, dist_loss, pos_out, negb_out, negm_out)


# speedup vs baseline: 1.3251x; 1.0607x over previous
import functools

import jax
import jax.numpy as jnp
from jax import lax
from jax.experimental import pallas as pl
from jax.experimental.pallas import tpu as pltpu
from jax.experimental.pallas import tpu_sc as plsc


def _dist_gather_sc(ctsd_flat, ct_flat, B):
    NW, LANES = 32, 16
    CH = B // NW
    mesh = plsc.VectorSubcoreMesh(core_axis_name="c", subcore_axis_name="s")

    @functools.partial(
        pl.kernel, mesh=mesh,
        out_type=[jax.ShapeDtypeStruct((B,), jnp.float32),
                  jax.ShapeDtypeStruct((B,), jnp.float32)],
        scratch_types=[pltpu.VMEM((CH,), jnp.int32),
                       pltpu.VMEM((CH,), jnp.int32),
                       pltpu.VMEM((CH,), jnp.int32),
                       pltpu.VMEM((CH,), jnp.float32),
                       pltpu.VMEM((CH,), jnp.float32),
                       pltpu.SemaphoreType.DMA,
                       pltpu.SemaphoreType.DMA],
    )
    def gather_kernel(ctsd_hbm, ct_hbm, g_out, s0_out,
                      ct_v, i0_v, ig_v, g_v, s0_v, sem1, sem2):
        wid = lax.axis_index("s") * 2 + lax.axis_index("c")
        base = wid * CH
        pltpu.sync_copy(ct_hbm.at[pl.ds(base, CH)], ct_v)

        def jbody(j, carry):
            pid16 = lax.broadcasted_iota(jnp.int32, (LANES,), 0) \
                + (base + j * LANES)
            i0 = pid16 * 81
            i0_v[pl.ds(j * LANES, LANES)] = i0
            ig_v[pl.ds(j * LANES, LANES)] = i0 + ct_v[pl.ds(j * LANES, LANES)]
            return carry

        lax.fori_loop(0, CH // LANES, jbody, 0)
        pltpu.async_copy(ctsd_hbm.at[ig_v], g_v, sem1).wait()
        pltpu.async_copy(ctsd_hbm.at[i0_v], s0_v, sem2).wait()
        pltpu.sync_copy(g_v, g_out.at[pl.ds(base, CH)])
        pltpu.sync_copy(s0_v, s0_out.at[pl.ds(base, CH)])

    return gather_kernel(ctsd_flat, ct_flat)


def _match_body(t_ref, pr_ref, loc_ref, bin_ref,
                conf_ref, posf_ref, cebin_ref, lossl_ref, npos_ref,
                *, T, R, P):
    px = pr_ref[0]
    py = pr_ref[1]
    pw = pr_ref[2]
    ph = pr_ref[3]
    pxmin = px - pw * 0.5
    pymin = py - ph * 0.5
    pxmax = px + pw * 0.5
    pymax = py + ph * 0.5
    parea = pw * ph
    pid = (lax.broadcasted_iota(jnp.int32, (R, 128), 0) * 128
           + lax.broadcasted_iota(jnp.int32, (R, 128), 1))

    best_ov = jnp.full((R, 128), -1.0, dtype=jnp.float32)
    best_idx = jnp.zeros((R, 128), dtype=jnp.int32)
    xs1, ys1, xs2, ys2, labs, bps = [], [], [], [], [], []
    for t in range(T):
        x1 = t_ref[0, t, 0]
        y1 = t_ref[0, t, 1]
        x2 = t_ref[0, t, 2]
        y2 = t_ref[0, t, 3]
        lab = t_ref[0, t, 4]
        xs1.append(x1); ys1.append(y1); xs2.append(x2); ys2.append(y2)
        labs.append(lab)
        iw = jnp.maximum(jnp.minimum(x2, pxmax) - jnp.maximum(x1, pxmin), 0.0)
        ih = jnp.maximum(jnp.minimum(y2, pymax) - jnp.maximum(y1, pymin), 0.0)
        inter = iw * ih
        at = (x2 - x1) * (y2 - y1)
        ov = inter / (at + parea - inter)
        upd = ov > best_ov
        best_idx = jnp.where(upd, t, best_idx)
        best_ov = jnp.where(upd, ov, best_ov)
        m = jnp.max(ov)
        bps.append(jnp.min(jnp.where(ov == m, pid, P)))
    force_t = jnp.full((R, 128), -1, dtype=jnp.int32)
    for t in range(T):
        force_t = jnp.where(pid == bps[t], t, force_t)
    forced = force_t >= 0
    best_ov = jnp.where(forced, 2.0, best_ov)
    best_idx = jnp.where(forced, force_t, best_idx)
    conf_i = jnp.zeros((R, 128), dtype=jnp.int32)
    mx1 = jnp.zeros((R, 128), dtype=jnp.float32)
    my1 = jnp.zeros((R, 128), dtype=jnp.float32)
    mx2 = jnp.zeros((R, 128), dtype=jnp.float32)
    my2 = jnp.zeros((R, 128), dtype=jnp.float32)
    for t in range(T):
        sel = best_idx == t
        conf_i = jnp.where(sel, labs[t].astype(jnp.int32) + 1, conf_i)
        mx1 = jnp.where(sel, xs1[t], mx1)
        my1 = jnp.where(sel, ys1[t], my1)
        mx2 = jnp.where(sel, xs2[t], mx2)
        my2 = jnp.where(sel, ys2[t], my2)
    conf_i = jnp.where(best_ov < 0.5, 0, conf_i)
    pos = conf_i > 0
    posf = pos.astype(jnp.float32)
    conf_ref[0] = conf_i
    posf_ref[0] = posf
    npos_ref[0] = jnp.sum(posf).reshape(1, 1)

    g1 = ((mx1 + mx2) * 0.5 - px) / (0.1 * pw)
    g2 = ((my1 + my2) * 0.5 - py) / (0.1 * ph)
    g3 = jnp.log((mx2 - mx1) / pw) / 0.2
    g4 = jnp.log((my2 - my1) / ph) / 0.2
    acc = jnp.float32(0.0)
    for c, g in enumerate((g1, g2, g3, g4)):
        d = loc_ref[0, c] - g
        ad = jnp.abs(d)
        sl = jnp.where(ad < 1.0, 0.5 * d * d, ad - 0.5)
        acc = acc + jnp.sum(jnp.where(pos, sl, 0.0))
    lossl_ref[0] = acc.reshape(1, 1)

    b0 = bin_ref[0, 0]
    b1 = bin_ref[0, 1]
    m2 = jnp.maximum(b0, b1)
    lse = m2 + jnp.log(jnp.exp(b0 - m2) + jnp.exp(b1 - m2))
    cebin_ref[0] = lse - jnp.where(pos, b1, b0)


def _conf_body(x_ref, ct_ref, cec_ref, *, BP, C):
    x = x_ref[0]
    ct = ct_ref[0, 0]
    m = jnp.max(x, axis=1, keepdims=True)
    e = jnp.exp(x - m)
    lse = m + jnp.log(jnp.sum(e, axis=1, keepdims=True))
    cio = lax.broadcasted_iota(jnp.int32, (1, C), 1)
    oh = cio == ct
    tgt = jnp.sum(jnp.where(oh, x, 0.0), axis=1, keepdims=True)
    cec_ref[0, 0] = lse - tgt


def _mine_body(cb_ref, cc_ref, pf_ref, s0_ref, gath_ref, np_ref,
               nb_ref, nm_ref, lb_ref, lc_ref, nd_ref, pd_ref,
               *, N_IMG, R, P):
    posf = pf_ref[...]
    pos = posf > 0.5
    npos = np_ref[...]
    k = jnp.minimum(npos * 3.0, jnp.float32(P - 1)).astype(jnp.int32)
    ceb = cb_ref[...]
    cec = cc_ref[...]

    ir = lax.broadcasted_iota(jnp.int32, (R, R), 0)
    jr = lax.broadcasted_iota(jnp.int32, (R, R), 1)
    UsR = (ir < jr).astype(jnp.float32)
    ia = lax.broadcasted_iota(jnp.int32, (128, 128), 0)
    ja = lax.broadcasted_iota(jnp.int32, (128, 128), 1)
    Us = (ia < ja).astype(jnp.float32)

    def per_image_sum(x):
        return jnp.sum(jnp.sum(x, axis=2), axis=1, keepdims=True)[:, :, None]

    def mine(ce):
        loss = jnp.maximum(jnp.where(pos, 0.0, ce), 0.0)
        keys = lax.bitcast_convert_type(loss, jnp.int32)

        def body(_, carry):
            lo, hi = carry
            d = hi - lo
            mid = lo + (d >> 1) + (d & 1)
            cnt = per_image_sum((keys >= mid).astype(jnp.int32))
            pred = cnt >= k
            return (jnp.where(pred, mid, lo), jnp.where(pred, hi, mid - 1))

        lo0 = jnp.zeros((N_IMG, 1, 1), jnp.int32)
        hi0 = jnp.full((N_IMG, 1, 1), 2**31 - 1, jnp.int32)
        lo, _ = lax.fori_loop(0, 31, body, (lo0, hi0))
        gt = keys > lo
        cgt = per_image_sum(gt.astype(jnp.int32))
        eq = keys == lo
        eqf = eq.astype(jnp.float32)
        rows = jnp.sum(eqf, axis=2)
        row_off = jnp.dot(rows, UsR, preferred_element_type=jnp.float32)
        lane_cum = lax.dot_general(eqf, Us, (((2,), (0,)), ((), ())),
                                   preferred_element_type=jnp.float32)
        cum = row_off[:, :, None] + lane_cum
        extra = (k - cgt).astype(jnp.float32)
        return gt | (eq & (cum < extra))

    negb = mine(ceb)
    negm = mine(cec)
    nb_ref[...] = negb.astype(jnp.float32)
    nm_ref[...] = negm.astype(jnp.float32)
    lb_ref[...] = per_image_sum(jnp.where(pos | negb, ceb, 0.0))
    lc_ref[...] = per_image_sum(jnp.where(pos | negm, cec, 0.0))
    nd_ref[...] = per_image_sum(jnp.where(negm, s0_ref[...], 0.0))
    pd_ref[...] = per_image_sum(jnp.where(pos, gath_ref[...], 0.0))


def kernel(loc_data, conf_data, bin_conf_data, priors, targets,
           conf_target_stand_dist):
    import functools
    N_IMG, P, C = conf_data.shape
    T = targets.shape[1]
    R = P // 128
    BP = min(4096, P)
    NB = P // BP

    pr_t = priors.T.reshape(4, R, 128)
    loc_t = loc_data.transpose(0, 2, 1).reshape(N_IMG, 4, R, 128)
    bin_t = bin_conf_data.transpose(0, 2, 1).reshape(N_IMG, 2, R, 128)

    f32 = jnp.float32
    conf_i, posf, cebin, lossl, nposs = pl.pallas_call(
        functools.partial(_match_body, T=T, R=R, P=P),
        grid=(N_IMG,),
        in_specs=[
            pl.BlockSpec((1, T, 5), lambda i: (i, 0, 0),
                         memory_space=pltpu.SMEM),
            pl.BlockSpec((4, R, 128), lambda i: (0, 0, 0)),
            pl.BlockSpec((1, 4, R, 128), lambda i: (i, 0, 0, 0)),
            pl.BlockSpec((1, 2, R, 128), lambda i: (i, 0, 0, 0)),
        ],
        out_specs=[
            pl.BlockSpec((1, R, 128), lambda i: (i, 0, 0)),
            pl.BlockSpec((1, R, 128), lambda i: (i, 0, 0)),
            pl.BlockSpec((1, R, 128), lambda i: (i, 0, 0)),
            pl.BlockSpec((1, 1, 1), lambda i: (i, 0, 0)),
            pl.BlockSpec((1, 1, 1), lambda i: (i, 0, 0)),
        ],
        out_shape=[
            jax.ShapeDtypeStruct((N_IMG, R, 128), jnp.int32),
            jax.ShapeDtypeStruct((N_IMG, R, 128), f32),
            jax.ShapeDtypeStruct((N_IMG, R, 128), f32),
            jax.ShapeDtypeStruct((N_IMG, 1, 1), f32),
            jax.ShapeDtypeStruct((N_IMG, 1, 1), f32),
        ],
    )(targets, pr_t, loc_t, bin_t)

    ct4 = conf_i.reshape(N_IMG, NB, BP, 1)
    gath_flat, s0_flat = _dist_gather_sc(
        conf_target_stand_dist.reshape(-1), conf_i.reshape(-1), N_IMG * P)

    cec4 = pl.pallas_call(
        functools.partial(_conf_body, BP=BP, C=C),
        grid=(N_IMG, NB),
        in_specs=[
            pl.BlockSpec((1, BP, C), lambda n, j: (n, j, 0)),
            pl.BlockSpec((1, 1, BP, 1), lambda n, j: (n, j, 0, 0)),
        ],
        out_specs=pl.BlockSpec((1, 1, BP, 1), lambda n, j: (n, j, 0, 0)),
        out_shape=jax.ShapeDtypeStruct((N_IMG, NB, BP, 1), f32),
    )(conf_data, ct4)

    cec = cec4.reshape(N_IMG, R, 128)
    s0 = s0_flat.reshape(N_IMG, R, 128)
    gath = gath_flat.reshape(N_IMG, R, 128)

    nbf, nmf, lb, lc, nd, pd = pl.pallas_call(
        functools.partial(_mine_body, N_IMG=N_IMG, R=R, P=P),
        out_shape=[
            jax.ShapeDtypeStruct((N_IMG, R, 128), f32),
            jax.ShapeDtypeStruct((N_IMG, R, 128), f32),
            jax.ShapeDtypeStruct((N_IMG, 1, 1), f32),
            jax.ShapeDtypeStruct((N_IMG, 1, 1), f32),
            jax.ShapeDtypeStruct((N_IMG, 1, 1), f32),
            jax.ShapeDtypeStruct((N_IMG, 1, 1), f32),
        ],
    )(cebin, cec, posf, s0, gath, nposs)

    N = jnp.maximum(jnp.sum(nposs), 1.0)
    loss_l = jnp.sum(lossl) / N
    loss_c = jnp.sum(lc) / N
    loss_bin = jnp.sum(lb) / N
    dist_loss = (-jnp.sum(pd) + 0.2 * jnp.sum(nd)) / N
    pos_out = posf.reshape(N_IMG, P) > 0.5
    negb_out = nbf.reshape(N_IMG, P) > 0.5
    negm_out = nmf.reshape(N_IMG, P) > 0.5
    return (loss_l, loss_c, loss_bin, dist_loss, pos_out, negb_out, negm_out)
